# Initial kernel scaffold; baseline (speedup 1.0000x reference)
import jax
import jax.numpy as jnp
from jax.experimental import pallas as pl


def kernel(x, table):
    raise NotImplementedError("write your pallas kernel here")


# speedup vs baseline: 1.0223x; 1.0223x over previous
import functools

import jax
import jax.numpy as jnp
from jax import lax
from jax.experimental import pallas as pl
from jax.experimental.pallas import tpu as pltpu
from jax.experimental.pallas import tpu_sc as plsc

EMBED = 32
GROUP = 128


def kernel(x, table):
    B = x.shape[0] * x.shape[1]
    assert B % GROUP == 0
    n_groups = B // GROUP

    info = plsc.get_sparse_core_info()
    NC, NS = info.num_cores, info.num_subcores
    NW = NC * NS
    assert n_groups % NW == 0
    g_per_w = n_groups // NW

    idx2d = x.reshape(n_groups, GROUP)

    mesh = plsc.VectorSubcoreMesh(core_axis_name="c", subcore_axis_name="s")

    @functools.partial(
        pl.kernel,
        mesh=mesh,
        compiler_params=pltpu.CompilerParams(use_tc_tiling_on_sc=False),
        out_type=jax.ShapeDtypeStruct((B, EMBED), jnp.float32),
        scratch_types=[
            pltpu.VMEM((g_per_w, GROUP), jnp.int32),
            pltpu.VMEM((GROUP, EMBED), jnp.float32),
            pltpu.SemaphoreType.DMA,
        ],
    )
    def emb(idx_hbm, table_hbm, out_hbm, idx_v, rows_v, sem):
        wid = lax.axis_index("s") * NC + lax.axis_index("c")
        gbase = wid * g_per_w
        pltpu.sync_copy(idx_hbm.at[pl.ds(gbase, g_per_w)], idx_v)

        def body(j, carry):
            pltpu.async_copy(table_hbm.at[idx_v.at[j]], rows_v, sem).wait()
            pltpu.sync_copy(
                rows_v, out_hbm.at[pl.ds((gbase + j) * GROUP, GROUP)]
            )
            return carry

        lax.fori_loop(0, g_per_w, body, 0)

    out = emb(idx2d, table)
    return out.reshape(x.shape[0], x.shape[1], EMBED)


# speedup vs baseline: 1.1088x; 1.0847x over previous
import functools

import jax
import jax.numpy as jnp
from jax import lax
from jax.experimental import pallas as pl
from jax.experimental.pallas import tpu as pltpu
from jax.experimental.pallas import tpu_sc as plsc

EMBED = 32
GROUP = 128
SG = 8


def kernel(x, table):
    B = x.shape[0] * x.shape[1]
    assert B % GROUP == 0
    n_groups = B // GROUP

    info = plsc.get_sparse_core_info()
    NC, NS = info.num_cores, info.num_subcores
    NW = NC * NS
    assert n_groups % (NW * SG) == 0
    g_per_w = n_groups // NW
    n_sg = g_per_w // SG
    rows_sg = SG * GROUP

    idx2d = x.reshape(n_groups, GROUP)

    mesh = plsc.VectorSubcoreMesh(core_axis_name="c", subcore_axis_name="s")

    @functools.partial(
        pl.kernel,
        mesh=mesh,
        compiler_params=pltpu.CompilerParams(use_tc_tiling_on_sc=False),
        out_type=jax.ShapeDtypeStruct((B, EMBED), jnp.float32),
        scratch_types=[
            pltpu.VMEM((g_per_w, GROUP), jnp.int32),
            pltpu.VMEM((2, rows_sg, EMBED), jnp.float32),
            pltpu.SemaphoreType.DMA,
            pltpu.SemaphoreType.DMA((2,)),
        ],
    )
    def emb(idx_hbm, table_hbm, out_hbm, idx_v, rows_v, gsem, wsem):
        wid = lax.axis_index("s") * NC + lax.axis_index("c")
        gbase = wid * g_per_w
        rbase = gbase * GROUP
        pltpu.sync_copy(idx_hbm.at[pl.ds(gbase, g_per_w)], idx_v)

        def body(i, carry):
            p = lax.rem(i, 2)

            @pl.when(i >= 2)
            def _wait_prev_writeback():
                pltpu.make_async_copy(
                    rows_v.at[p],
                    out_hbm.at[pl.ds(rbase, rows_sg)],
                    wsem.at[p],
                ).wait()

            descs = []
            for b in range(SG):
                descs.append(
                    pltpu.async_copy(
                        table_hbm.at[idx_v.at[i * SG + b]],
                        rows_v.at[p, pl.ds(b * GROUP, GROUP)],
                        gsem,
                    )
                )
            for d in descs:
                d.wait()

            pltpu.async_copy(
                rows_v.at[p],
                out_hbm.at[pl.ds(rbase + i * rows_sg, rows_sg)],
                wsem.at[p],
            )
            return carry

        lax.fori_loop(0, n_sg, body, 0)

        for p in range(2):
            pltpu.make_async_copy(
                rows_v.at[p],
                out_hbm.at[pl.ds(rbase, rows_sg)],
                wsem.at[p],
            ).wait()

    out = emb(idx2d, table)
    return out.reshape(x.shape[0], x.shape[1], EMBED)


# speedup vs baseline: 1.8658x; 1.6826x over previous
import functools

import jax
import jax.numpy as jnp
from jax import lax
from jax.experimental import pallas as pl
from jax.experimental.pallas import tpu as pltpu
from jax.experimental.pallas import tpu_sc as plsc

EMBED = 32
GROUP = 128
SETB = 4
BLKW = EMBED * GROUP
SETW = SETB * BLKW


def kernel(x, table):
    NI, NJ = x.shape
    assert NI % GROUP == 0
    n_blocks = NJ * (NI // GROUP)
    ib_per_j = NI // GROUP

    info = plsc.get_sparse_core_info()
    NC, NS = info.num_cores, info.num_subcores
    NW = NC * NS
    assert n_blocks % (NW * SETB) == 0
    g_per_w = n_blocks // NW
    n_sets = g_per_w // SETB
    assert n_sets % 2 == 0
    n_pairs = n_sets // 2

    idx2d = jnp.transpose(x).reshape(n_blocks, GROUP)
    out_words = NJ * EMBED * NI

    mesh = plsc.VectorSubcoreMesh(core_axis_name="c", subcore_axis_name="s")

    @functools.partial(
        pl.kernel,
        mesh=mesh,
        compiler_params=pltpu.CompilerParams(
            use_tc_tiling_on_sc=False, needs_layout_passes=False
        ),
        out_type=jax.ShapeDtypeStruct((out_words,), jnp.float32),
        scratch_types=[
            pltpu.VMEM((g_per_w, GROUP), jnp.int32),
            pltpu.VMEM((SETB, GROUP, EMBED), jnp.float32),
            pltpu.VMEM((SETB, GROUP, EMBED), jnp.float32),
            pltpu.VMEM((SETW,), jnp.float32),
            pltpu.VMEM((SETW,), jnp.float32),
            pltpu.SemaphoreType.DMA,
            pltpu.SemaphoreType.DMA,
            pltpu.SemaphoreType.DMA,
            pltpu.SemaphoreType.DMA,
        ],
    )
    def emb(idx_hbm, table_hbm, out_hbm, idx_v, rows0, rows1, blk0, blk1,
            gsem0, gsem1, wsem0, wsem1):
        wid = lax.axis_index("s") * NC + lax.axis_index("c")
        gbase = wid * g_per_w
        pltpu.sync_copy(idx_hbm.at[pl.ds(gbase, g_per_w)], idx_v)

        lane = lax.iota(jnp.int32, 16)
        b_lo = (lane >> 3) * BLKW + (lane & 7) * GROUP

        def fire_set(s, rows_p, gsem_p):
            for b in range(SETB):
                pltpu.async_copy(
                    table_hbm.at[idx_v.at[s * SETB + b]],
                    rows_p.at[b],
                    gsem_p,
                )

        def drain_set(rows_p, gsem_p):
            for b in range(SETB):
                pltpu.make_async_copy(
                    table_hbm.at[pl.ds(0, GROUP)], rows_p.at[b], gsem_p
                ).wait()

        def do_set(it, p, rows_p, rows_q, blk_p, gsem_p, gsem_q, wsem_p):
            s = it * 2 + p
            g0 = gbase + s * SETB

            if p == 0:
                fire_set(s + 1, rows_q, gsem_q)
            else:
                @pl.when(it < n_pairs - 1)
                def _fire_next():
                    fire_set(s + 1, rows_q, gsem_q)

            drain_set(rows_p, gsem_p)

            @pl.when(it >= 1)
            def _wait_prev_writeback():
                pltpu.make_async_copy(
                    blk_p, out_hbm.at[pl.ds(0, SETW)], wsem_p
                ).wait()

            for b in range(SETB):
                def tbody(i2, carry):
                    off = b * (8 * GROUP) + i2
                    d_lo = rows_p[b, i2, pl.ds(0, 16)]
                    d_hi = rows_p[b, i2, pl.ds(16, 16)]
                    plsc.store_scatter(blk_p, [b_lo + off], d_lo)
                    plsc.store_scatter(blk_p, [b_lo + off + 2 * BLKW], d_hi)
                    return carry

                lax.fori_loop(0, GROUP, tbody, 0)

            j = g0 // ib_per_j
            ib0 = g0 % ib_per_j
            for eb in range(EMBED // 8):
                off = ((j * (EMBED // 8) + eb) * ib_per_j + ib0) * (8 * GROUP)
                pltpu.async_copy(
                    blk_p.at[pl.ds(eb * BLKW, BLKW)],
                    out_hbm.at[pl.ds(off, BLKW)],
                    wsem_p,
                )

        fire_set(0, rows0, gsem0)

        def body(it, carry):
            do_set(it, 0, rows0, rows1, blk0, gsem0, gsem1, wsem0)
            do_set(it, 1, rows1, rows0, blk1, gsem1, gsem0, wsem1)
            return carry

        lax.fori_loop(0, n_pairs, body, 0)

        pltpu.make_async_copy(blk0, out_hbm.at[pl.ds(0, SETW)], wsem0).wait()
        pltpu.make_async_copy(blk1, out_hbm.at[pl.ds(0, SETW)], wsem1).wait()

    out1d = emb(idx2d, table)
    return (
        out1d.reshape(NJ, EMBED // 8, ib_per_j, 8, GROUP)
        .transpose((2, 4, 0, 1, 3))
        .reshape(NI, NJ, EMBED)
    )
